# Initial kernel scaffold; baseline (speedup 1.0000x reference)
import jax
import jax.numpy as jnp
from jax.experimental import pallas as pl


def kernel(hidden_states, W):
    raise NotImplementedError("write your pallas kernel here")


# speedup vs baseline: 1.1629x; 1.1629x over previous
import functools

import jax
import jax.numpy as jnp
from jax.experimental import pallas as pl

B, S, H, E, K = 4, 2048, 1024, 8, 2
CAP = (S * K) // E


def _router_body(hs_ref, wt_ref, disp_ref, comb_ref, probs_ref, aux_ref, z_ref):
    hs = hs_ref[0]
    wt = wt_ref[...]
    logits = jnp.dot(hs, wt, preferred_element_type=jnp.float32)

    m = jnp.max(logits, axis=-1, keepdims=True)
    el = jnp.exp(logits - m)
    sel = jnp.sum(el, axis=-1, keepdims=True)
    probs = el / sel
    probs_ref[0] = probs

    lse = m + jnp.log(sel)
    z_ref[...] = jnp.sum(lse * lse).reshape(1, 1, 1)
    aux_ref[...] = jnp.sum(probs * probs).reshape(1, 1, 1)

    eidx = jax.lax.broadcasted_iota(jnp.int32, (S, E), 1)
    m1 = jnp.max(probs, axis=-1, keepdims=True)
    i1 = jnp.min(jnp.where(probs == m1, eidx, E), axis=-1, keepdims=True)
    p2 = jnp.where(eidx == i1, -1.0, probs)
    m2 = jnp.max(p2, axis=-1, keepdims=True)
    i2 = jnp.min(jnp.where(p2 == m2, eidx, E), axis=-1, keepdims=True)

    wsum = m1 + m2
    w1 = m1 / wsum
    w2 = m2 / wsum

    oh1 = (eidx == i1).astype(jnp.int32)
    oh2 = (eidx == i2).astype(jnp.int32)
    c0 = jnp.sum(oh1, axis=0, keepdims=True)

    x = jnp.concatenate([oh1, oh2], axis=1)
    d = 1
    while d < S:
        shifted = jnp.concatenate(
            [jnp.zeros((d, 2 * E), jnp.int32), x[: S - d]], axis=0)
        x = x + shifted
        d *= 2
    r1 = x[:, :E] - oh1
    r2 = x[:, E:] - oh2 + c0

    a1 = ((oh1 > 0) & (r1 < CAP)).astype(jnp.float32)
    a2 = ((oh2 > 0) & (r2 < CAP)).astype(jnp.float32)
    disp_ref[0] = a1 + a2
    comb_ref[0] = a1 * w1 + a2 * w2


@functools.partial(jax.jit, static_argnames=())
def kernel(hidden_states, W):
    wt = W.T
    disp, comb, probs, aux, z = pl.pallas_call(
        _router_body,
        grid=(B,),
        in_specs=[
            pl.BlockSpec((1, S, H), lambda b: (b, 0, 0)),
            pl.BlockSpec((H, E), lambda b: (0, 0)),
        ],
        out_specs=[
            pl.BlockSpec((1, S, E), lambda b: (b, 0, 0)),
            pl.BlockSpec((1, S, E), lambda b: (b, 0, 0)),
            pl.BlockSpec((1, S, E), lambda b: (b, 0, 0)),
            pl.BlockSpec((1, 1, 1), lambda b: (b, 0, 0)),
            pl.BlockSpec((1, 1, 1), lambda b: (b, 0, 0)),
        ],
        out_shape=[
            jax.ShapeDtypeStruct((B, S, E), jnp.float32),
            jax.ShapeDtypeStruct((B, S, E), jnp.float32),
            jax.ShapeDtypeStruct((B, S, E), jnp.float32),
            jax.ShapeDtypeStruct((B, 1, 1), jnp.float32),
            jax.ShapeDtypeStruct((B, 1, 1), jnp.float32),
        ],
    )(hidden_states, wt)
    aux_loss = (jnp.sum(aux) / (B * S)) * E
    z_loss = jnp.sum(z) / (B * S)
    return (disp, comb, probs, aux_loss.reshape(()), z_loss.reshape(()))
